# Initial kernel scaffold; baseline (speedup 1.0000x reference)
import jax
import jax.numpy as jnp
from jax.experimental import pallas as pl


def kernel(x, table):
    raise NotImplementedError("write your pallas kernel here")


# speedup vs baseline: 9.2391x; 9.2391x over previous
import functools

import jax
import jax.numpy as jnp
from jax import lax
from jax.experimental import pallas as pl
from jax.experimental.pallas import tpu as pltpu
from jax.experimental.pallas import tpu_sc as plsc

NC = 2
NS = 16
NW = NC * NS
CH = 128


def _make_gather(B, V, D):
  n_chunks = B // (NW * CH)
  assert B % (NW * CH) == 0 and n_chunks % 2 == 0

  mesh = plsc.VectorSubcoreMesh(
      core_axis_name="c", subcore_axis_name="s", num_cores=NC, num_subcores=NS
  )

  @functools.partial(
      pl.kernel,
      mesh=mesh,
      out_type=jax.ShapeDtypeStruct((B, D), jnp.float32),
      scratch_types=[
          pltpu.VMEM((n_chunks, CH), jnp.int32),
          pltpu.VMEM((CH, D), jnp.float32),
          pltpu.VMEM((CH, D), jnp.float32),
          pltpu.SemaphoreType.DMA,
          pltpu.SemaphoreType.DMA,
      ],
  )
  def gather(table_hbm, idx_hbm, out_hbm, idx_v, buf0, buf1, sem0, sem1):
    wid = lax.axis_index("s") * NC + lax.axis_index("c")
    base = wid * (n_chunks * CH)

    pltpu.sync_copy(idx_hbm.at[wid], idx_v)

    def gather_chunk(j, buf, sem):
      return pltpu.make_async_copy(table_hbm.at[idx_v.at[j]], buf, sem)

    gather_chunk(0, buf0, sem0).start()

    def body(g, _):
      j0 = 2 * g
      gather_chunk(j0 + 1, buf1, sem1).start()
      gather_chunk(0, buf0, sem0).wait()
      pltpu.sync_copy(buf0, out_hbm.at[pl.ds(base + j0 * CH, CH)])

      @pl.when(j0 + 2 < n_chunks)
      def _():
        gather_chunk(j0 + 2, buf0, sem0).start()

      gather_chunk(0, buf1, sem1).wait()
      pltpu.sync_copy(buf1, out_hbm.at[pl.ds(base + (j0 + 1) * CH, CH)])
      return 0

    lax.fori_loop(0, n_chunks // 2, body, 0)

  return gather


def kernel(x, table):
  B0, B1 = x.shape
  V, D = table.shape
  B = B0 * B1
  idx = x.reshape(NW, B // (NW * CH), CH).astype(jnp.int32)
  out = _make_gather(B, V, D)(table, idx)
  return out.reshape(B0, B1, D)
